# Initial kernel scaffold; baseline (speedup 1.0000x reference)
import jax
import jax.numpy as jnp
from jax.experimental import pallas as pl


def kernel(emb, edge_feat, timestamp, last_update, src_idx, time_w, time_b, Wq, bq, Wk, bk, Wv, bv, Wo, bo, W1, b1, W2, b2):
    raise NotImplementedError("write your pallas kernel here")


# speedup vs baseline: 3.3242x; 3.3242x over previous
import functools
import math

import jax
import jax.numpy as jnp
from jax import lax
from jax.experimental import pallas as pl
from jax.experimental.pallas import tpu as pltpu
from jax.experimental.pallas import tpu_sc as plsc

_N = 10000
_K = 32
_E = _N * _K
_DN = 128
_DE = 16
_DT = 128
_QD = 256
_H = 2

_GRP = 80
_GPI = 5
_CH = _GRP * _GPI

_B = 200
_BK = _B * _K


def _sc_gather(emb, last_update, idx1d):
    info = plsc.get_sparse_core_info()
    nw = info.num_cores * info.num_subcores
    per_w = _E // nw
    n_iter = per_w // _CH
    mesh = plsc.VectorSubcoreMesh(core_axis_name="c", subcore_axis_name="s")

    @functools.partial(
        pl.kernel,
        out_type=(
            jax.ShapeDtypeStruct((_E, _DN), jnp.float32),
            jax.ShapeDtypeStruct((_E,), jnp.float32),
        ),
        mesh=mesh,
        scratch_types=[
            pltpu.VMEM((_CH,), jnp.int32),
            pltpu.VMEM((_CH, _DN), jnp.float32),
            pltpu.VMEM((_CH,), jnp.float32),
            pltpu.SemaphoreType.DMA,
            pltpu.SemaphoreType.DMA,
        ],
    )
    def k(emb_hbm, lu_hbm, idx_hbm, se_out, lu_out, idx_v, rows_v, lus_v,
          sem_r, sem_l):
        wid = lax.axis_index("s") * info.num_cores + lax.axis_index("c")
        ebase = wid * per_w

        def body(t, _):
            e0 = ebase + t * _CH
            pltpu.sync_copy(idx_hbm.at[pl.ds(e0, _CH)], idx_v)
            cps = []
            for j in range(_GPI):
                gs = pl.ds(j * _GRP, _GRP)
                cps.append(pltpu.async_copy(
                    emb_hbm.at[idx_v.at[gs]], rows_v.at[gs], sem_r))
                cps.append(pltpu.async_copy(
                    lu_hbm.at[idx_v.at[gs]], lus_v.at[gs], sem_l))
            for cp in cps:
                cp.wait()
            pltpu.sync_copy(rows_v, se_out.at[pl.ds(e0, _CH)])
            pltpu.sync_copy(lus_v, lu_out.at[pl.ds(e0, _CH)])
            return 0

        lax.fori_loop(0, n_iter, body, 0)

    return k(emb, last_update, idx1d)


def _tc_body(se_ref, ef_ref, ts_ref, lu_ref, em_ref,
             tw_ref, tb_ref, wqn_ref, wqt_ref, bq_ref,
             wkn_ref, wke_ref, wkt_ref,
             wvn_ref, wve_ref, wvt_ref, bv_ref,
             wo_ref, bo_ref, w1_ref, b1_ref, w2_ref, b2_ref,
             out_ref):
    f32 = jnp.float32
    dot = functools.partial(jnp.dot, preferred_element_type=f32)
    se = se_ref[...]
    ef = ef_ref[...]
    dt = ts_ref[...] - lu_ref[...]
    tw = tw_ref[...]
    tb = tb_ref[...]
    te = jnp.cos(dt * tw + tb)
    em = em_ref[...]

    te0 = jnp.cos(tb)
    cq = dot(te0, wqt_ref[...]) + bq_ref[...]
    q = (dot(em, wqn_ref[...]) + cq) * (1.0 / math.sqrt(128.0))

    se3 = se.reshape(_B, _K, _DN)
    ef3 = ef.reshape(_B, _K, _DE)
    te3 = te.reshape(_B, _K, _DT)

    ctxs = []
    for h in range(_H):
        hs = slice(h * 128, (h + 1) * 128)
        qh = q[:, hs]
        qkn = dot(qh, wkn_ref[hs, :])
        qke = dot(qh, wke_ref[hs, :])
        qkt = dot(qh, wkt_ref[hs, :])
        lg = (jnp.sum(se3 * qkn[:, None, :], axis=-1)
              + jnp.sum(ef3 * qke[:, None, :], axis=-1)
              + jnp.sum(te3 * qkt[:, None, :], axis=-1))
        m = jnp.max(lg, axis=-1, keepdims=True)
        p = jnp.exp(lg - m)
        att = p / jnp.sum(p, axis=-1, keepdims=True)
        a3 = att[:, :, None]
        cn = jnp.sum(se3 * a3, axis=1)
        ce = jnp.sum(ef3 * a3, axis=1)
        ct = jnp.sum(te3 * a3, axis=1)
        ctx = (dot(cn, wvn_ref[:, hs]) + dot(ce, wve_ref[:, hs])
               + dot(ct, wvt_ref[:, hs]) + bv_ref[:, hs])
        ctxs.append(ctx)

    hb = (dot(ctxs[0], wo_ref[0:128, :]) + dot(ctxs[1], wo_ref[128:256, :])
          + bo_ref[...])
    x1 = dot(em, w1_ref[0:128, :]) + dot(hb, w1_ref[128:384, :]) + b1_ref[...]
    h1 = jnp.maximum(x1, 0.0)
    out_ref[...] = dot(h1, w2_ref[...]) + b2_ref[...]


def _tc_main(se, ef2, ts2, lu2, emb, tw2, tb2, wqnT, wqtT, bq2,
             wkn, wke, wkt, wvnT, wveT, wvtT, bv2,
             woT, bo2, w1T, b12, w2T, b22, interpret=False):
    full = lambda s: pl.BlockSpec(s, lambda i: (0, 0))
    grid = _N // _B
    return pl.pallas_call(
        _tc_body,
        grid=(grid,),
        in_specs=[
            pl.BlockSpec((_BK, _DN), lambda i: (i, 0)),
            pl.BlockSpec((_BK, _DE), lambda i: (i, 0)),
            pl.BlockSpec((_BK, 1), lambda i: (i, 0)),
            pl.BlockSpec((_BK, 1), lambda i: (i, 0)),
            pl.BlockSpec((_B, _DN), lambda i: (i, 0)),
            full((1, _DT)), full((1, _DT)),
            full((_DN, _QD)), full((_DT, _QD)), full((1, _QD)),
            full((_QD, _DN)), full((_QD, _DE)), full((_QD, _DT)),
            full((_DN, _QD)), full((_DE, _QD)), full((_DT, _QD)),
            full((1, _QD)),
            full((_QD, _QD)), full((1, _QD)),
            full((_QD + _DN, _DN)), full((1, _DN)),
            full((_DN, _DN)), full((1, _DN)),
        ],
        out_specs=pl.BlockSpec((_B, _DN), lambda i: (i, 0)),
        out_shape=jax.ShapeDtypeStruct((_N, _DN), jnp.float32),
        interpret=interpret,
    )(se, ef2, ts2, lu2, emb, tw2, tb2, wqnT, wqtT, bq2,
      wkn, wke, wkt, wvnT, wveT, wvtT, bv2, woT, bo2, w1T, b12, w2T, b22)


def kernel(emb, edge_feat, timestamp, last_update, src_idx, time_w, time_b,
           Wq, bq, Wk, bk, Wv, bv, Wo, bo, W1, b1, W2, b2):
    src_emb, lu_src = _sc_gather(emb, last_update, src_idx.astype(jnp.int32))

    ts2 = timestamp.reshape(_E, 1)
    lu2 = lu_src.reshape(_E, 1)
    tw2 = time_w.reshape(1, _DT)
    tb2 = time_b.reshape(1, _DT)
    wqnT = Wq[:, :_DN].T
    wqtT = Wq[:, _DN:].T
    wkn = Wk[:, :_DN]
    wke = Wk[:, _DN:_DN + _DE]
    wkt = Wk[:, _DN + _DE:]
    wvnT = Wv[:, :_DN].T
    wveT = Wv[:, _DN:_DN + _DE].T
    wvtT = Wv[:, _DN + _DE:].T
    return _tc_main(src_emb, edge_feat, ts2, lu2, emb, tw2, tb2,
                    wqnT, wqtT, bq.reshape(1, -1),
                    wkn, wke, wkt, wvnT, wveT, wvtT, bv.reshape(1, -1),
                    Wo.T, bo.reshape(1, -1), W1.T, b1.reshape(1, -1),
                    W2.T, b2.reshape(1, -1))


# speedup vs baseline: 4.6521x; 1.3995x over previous
import functools
import math

import jax
import jax.numpy as jnp
from jax import lax
from jax.experimental import pallas as pl
from jax.experimental.pallas import tpu as pltpu
from jax.experimental.pallas import tpu_sc as plsc

_N = 10000
_K = 32
_E = _N * _K
_DN = 128
_DE = 16
_DT = 128
_QD = 256
_H = 2

_GRP = 80
_GPI = 5
_CH = _GRP * _GPI

_B = 200
_BK = _B * _K


def _sc_gather(emb, last_update, idx1d):
    info = plsc.get_sparse_core_info()
    nw = info.num_cores * info.num_subcores
    per_w = _E // nw
    n_iter = per_w // _CH
    mesh = plsc.VectorSubcoreMesh(core_axis_name="c", subcore_axis_name="s")

    @functools.partial(
        pl.kernel,
        out_type=(
            jax.ShapeDtypeStruct((_E, _DN), jnp.float32),
            jax.ShapeDtypeStruct((_E,), jnp.float32),
        ),
        mesh=mesh,
        scratch_types=[
            pltpu.VMEM((_CH,), jnp.int32),
            pltpu.VMEM((_CH, _DN), jnp.float32),
            pltpu.VMEM((_CH,), jnp.float32),
            pltpu.SemaphoreType.DMA,
            pltpu.SemaphoreType.DMA,
        ],
    )
    def k(emb_hbm, lu_hbm, idx_hbm, se_out, lu_out, idx_v, rows_v, lus_v,
          sem_r, sem_l):
        wid = lax.axis_index("s") * info.num_cores + lax.axis_index("c")
        ebase = wid * per_w

        def body(t, _):
            e0 = ebase + t * _CH
            pltpu.sync_copy(idx_hbm.at[pl.ds(e0, _CH)], idx_v)
            cps = []
            for j in range(_GPI):
                gs = pl.ds(j * _GRP, _GRP)
                cps.append(pltpu.async_copy(
                    emb_hbm.at[idx_v.at[gs]], rows_v.at[gs], sem_r))
                cps.append(pltpu.async_copy(
                    lu_hbm.at[idx_v.at[gs]], lus_v.at[gs], sem_l))
            for cp in cps:
                cp.wait()
            pltpu.sync_copy(rows_v, se_out.at[pl.ds(e0, _CH)])
            pltpu.sync_copy(lus_v, lu_out.at[pl.ds(e0, _CH)])
            return 0

        lax.fori_loop(0, n_iter, body, 0)

    return k(emb, last_update, idx1d)


_INV2PI = 0.15915494309189535
_COSC = (0.999999210881232, -19.738980362141934, 64.92865752951663,
         -85.2716222117908, 58.790493572670144, -21.071105911444665)


def _fast_cos(x):
    u = x * _INV2PI
    u = u - jnp.round(u)
    t = u * u
    r = jnp.float32(_COSC[5])
    for c in _COSC[4::-1]:
        r = r * t + jnp.float32(c)
    return r


def _tc_body(se_ref, ef_ref, ts_ref, lu_ref, em_ref,
             tw_ref, tb_ref, wqn_ref, wqt_ref, bq_ref,
             wkn_ref, wke_ref, wkt_ref,
             wvn_ref, wve_ref, wvt_ref, bv_ref,
             wo_ref, bo_ref, w1_ref, b1_ref, w2_ref, b2_ref,
             out_ref):
    f32 = jnp.float32
    dot = functools.partial(jnp.dot, preferred_element_type=f32)
    se = se_ref[...]
    ef = ef_ref[...]
    dt = ts_ref[...] - lu_ref[...]
    tw = tw_ref[...]
    tb = tb_ref[...]
    te = _fast_cos(dt * tw + tb)
    em = em_ref[...]

    te0 = jnp.cos(tb)
    cq = dot(te0, wqt_ref[...]) + bq_ref[...]
    q = (dot(em, wqn_ref[...]) + cq) * (1.0 / math.sqrt(128.0))

    se3 = se.reshape(_B, _K, _DN)
    ef3 = ef.reshape(_B, _K, _DE)
    te3 = te.reshape(_B, _K, _DT)

    ctxs = []
    for h in range(_H):
        hs = slice(h * 128, (h + 1) * 128)
        qh = q[:, hs]
        qkn = dot(qh, wkn_ref[hs, :])
        qke = dot(qh, wke_ref[hs, :])
        qkt = dot(qh, wkt_ref[hs, :])
        lg = (jnp.sum(se3 * qkn[:, None, :], axis=-1)
              + jnp.sum(ef3 * qke[:, None, :], axis=-1)
              + jnp.sum(te3 * qkt[:, None, :], axis=-1))
        m = jnp.max(lg, axis=-1, keepdims=True)
        p = jnp.exp(lg - m)
        att = p / jnp.sum(p, axis=-1, keepdims=True)
        a3 = att[:, :, None]
        cn = jnp.sum(se3 * a3, axis=1)
        ce = jnp.sum(ef3 * a3, axis=1)
        ct = jnp.sum(te3 * a3, axis=1)
        ctx = (dot(cn, wvn_ref[:, hs]) + dot(ce, wve_ref[:, hs])
               + dot(ct, wvt_ref[:, hs]) + bv_ref[:, hs])
        ctxs.append(ctx)

    hb = (dot(ctxs[0], wo_ref[0:128, :]) + dot(ctxs[1], wo_ref[128:256, :])
          + bo_ref[...])
    x1 = dot(em, w1_ref[0:128, :]) + dot(hb, w1_ref[128:384, :]) + b1_ref[...]
    h1 = jnp.maximum(x1, 0.0)
    out_ref[...] = dot(h1, w2_ref[...]) + b2_ref[...]


def _tc_main(se, ef2, ts2, lu2, emb, tw2, tb2, wqnT, wqtT, bq2,
             wkn, wke, wkt, wvnT, wveT, wvtT, bv2,
             woT, bo2, w1T, b12, w2T, b22, interpret=False):
    full = lambda s: pl.BlockSpec(s, lambda i: (0, 0))
    grid = _N // _B
    return pl.pallas_call(
        _tc_body,
        grid=(grid,),
        in_specs=[
            pl.BlockSpec((_BK, _DN), lambda i: (i, 0)),
            pl.BlockSpec((_BK, _DE), lambda i: (i, 0)),
            pl.BlockSpec((_BK, 1), lambda i: (i, 0)),
            pl.BlockSpec((_BK, 1), lambda i: (i, 0)),
            pl.BlockSpec((_B, _DN), lambda i: (i, 0)),
            full((1, _DT)), full((1, _DT)),
            full((_DN, _QD)), full((_DT, _QD)), full((1, _QD)),
            full((_QD, _DN)), full((_QD, _DE)), full((_QD, _DT)),
            full((_DN, _QD)), full((_DE, _QD)), full((_DT, _QD)),
            full((1, _QD)),
            full((_QD, _QD)), full((1, _QD)),
            full((_QD + _DN, _DN)), full((1, _DN)),
            full((_DN, _DN)), full((1, _DN)),
        ],
        out_specs=pl.BlockSpec((_B, _DN), lambda i: (i, 0)),
        out_shape=jax.ShapeDtypeStruct((_N, _DN), jnp.float32),
        interpret=interpret,
    )(se, ef2, ts2, lu2, emb, tw2, tb2, wqnT, wqtT, bq2,
      wkn, wke, wkt, wvnT, wveT, wvtT, bv2, woT, bo2, w1T, b12, w2T, b22)


def kernel(emb, edge_feat, timestamp, last_update, src_idx, time_w, time_b,
           Wq, bq, Wk, bk, Wv, bv, Wo, bo, W1, b1, W2, b2):
    src_emb, lu_src = _sc_gather(emb, last_update, src_idx.astype(jnp.int32))

    ts2 = timestamp.reshape(_E, 1)
    lu2 = lu_src.reshape(_E, 1)
    tw2 = time_w.reshape(1, _DT)
    tb2 = time_b.reshape(1, _DT)
    wqnT = Wq[:, :_DN].T
    wqtT = Wq[:, _DN:].T
    wkn = Wk[:, :_DN]
    wke = Wk[:, _DN:_DN + _DE]
    wkt = Wk[:, _DN + _DE:]
    wvnT = Wv[:, :_DN].T
    wveT = Wv[:, _DN:_DN + _DE].T
    wvtT = Wv[:, _DN + _DE:].T
    return _tc_main(src_emb, edge_feat, ts2, lu2, emb, tw2, tb2,
                    wqnT, wqtT, bq.reshape(1, -1),
                    wkn, wke, wkt, wvnT, wveT, wvtT, bv.reshape(1, -1),
                    Wo.T, bo.reshape(1, -1), W1.T, b1.reshape(1, -1),
                    W2.T, b2.reshape(1, -1))


# speedup vs baseline: 5.8431x; 1.2560x over previous
import functools
import math

import jax
import jax.numpy as jnp
from jax import lax
from jax.experimental import pallas as pl
from jax.experimental.pallas import tpu as pltpu
from jax.experimental.pallas import tpu_sc as plsc

_N = 10000
_K = 32
_E = _N * _K
_DN = 128
_DE = 16
_DT = 128
_QD = 256
_H = 2

_GRP = 80
_GPI = 5
_CH = _GRP * _GPI

_B = 200
_BK = _B * _K


def _sc_gather(emb, last_update, idx1d):
    info = plsc.get_sparse_core_info()
    nw = info.num_cores * info.num_subcores
    per_w = _E // nw
    n_iter = per_w // _CH
    mesh = plsc.VectorSubcoreMesh(core_axis_name="c", subcore_axis_name="s")

    @functools.partial(
        pl.kernel,
        out_type=(
            jax.ShapeDtypeStruct((_E, _DN), jnp.float32),
            jax.ShapeDtypeStruct((_E,), jnp.float32),
        ),
        mesh=mesh,
        scratch_types=[
            pltpu.VMEM((_CH,), jnp.int32),
            pltpu.VMEM((_CH, _DN), jnp.float32),
            pltpu.VMEM((_CH,), jnp.float32),
            pltpu.SemaphoreType.DMA,
            pltpu.SemaphoreType.DMA,
        ],
    )
    def k(emb_hbm, lu_hbm, idx_hbm, se_out, lu_out, idx_v, rows_v, lus_v,
          sem_r, sem_l):
        wid = lax.axis_index("s") * info.num_cores + lax.axis_index("c")
        ebase = wid * per_w

        def body(t, _):
            e0 = ebase + t * _CH
            pltpu.sync_copy(idx_hbm.at[pl.ds(e0, _CH)], idx_v)
            cps = []
            for j in range(_GPI):
                gs = pl.ds(j * _GRP, _GRP)
                cps.append(pltpu.async_copy(
                    emb_hbm.at[idx_v.at[gs]], rows_v.at[gs], sem_r))
                cps.append(pltpu.async_copy(
                    lu_hbm.at[idx_v.at[gs]], lus_v.at[gs], sem_l))
            for cp in cps:
                cp.wait()
            pltpu.sync_copy(rows_v, se_out.at[pl.ds(e0, _CH)])
            pltpu.sync_copy(lus_v, lu_out.at[pl.ds(e0, _CH)])
            return 0

        lax.fori_loop(0, n_iter, body, 0)

    return k(emb, last_update, idx1d)


_INV2PI = 0.15915494309189535
_COSC = (0.999999210881232, -19.738980362141934, 64.92865752951663,
         -85.2716222117908, 58.790493572670144, -21.071105911444665)


def _fast_cos(x):
    u = x * _INV2PI
    u = u - jnp.round(u)
    t = u * u
    r = jnp.float32(_COSC[5])
    for c in _COSC[4::-1]:
        r = r * t + jnp.float32(c)
    return r


def _tc_body(se_ref, ef_ref, ts_ref, lu_ref, em_ref,
             tw_ref, tb_ref, wqn_ref, wqt_ref, bq_ref,
             wkn_ref, wke_ref, wkt_ref,
             wvn_ref, wve_ref, wvt_ref, bv_ref,
             wo_ref, bo_ref, w1_ref, b1_ref, w2_ref, b2_ref,
             out_ref):
    f32 = jnp.float32
    dot = functools.partial(jnp.dot, preferred_element_type=f32)
    se = se_ref[...]
    ef = ef_ref[...]
    dt = (ts_ref[...] - lu_ref[...]).reshape(_BK // 128, 128)
    tw3 = tw_ref[...].reshape(1, 1, _DT)
    tb3 = tb_ref[...].reshape(1, 1, _DT)
    dt3 = dt[:, :, None]
    te = _fast_cos(dt3 * tw3 + tb3).reshape(_BK, _DT)
    tb = tb_ref[...]
    em = em_ref[...]

    te0 = jnp.cos(tb)
    cq = dot(te0, wqt_ref[...]) + bq_ref[...]
    q = (dot(em, wqn_ref[...]) + cq) * (1.0 / math.sqrt(128.0))

    se3 = se.reshape(_B, _K, _DN)
    ef3 = ef.reshape(_B, _K, _DE)
    te3 = te.reshape(_B, _K, _DT)

    ctxs = []
    for h in range(_H):
        hs = slice(h * 128, (h + 1) * 128)
        qh = q[:, hs]
        qkn = dot(qh, wkn_ref[hs, :])
        qke = dot(qh, wke_ref[hs, :])
        qkt = dot(qh, wkt_ref[hs, :])
        lg = (jnp.sum(se3 * qkn[:, None, :], axis=-1)
              + jnp.sum(ef3 * qke[:, None, :], axis=-1)
              + jnp.sum(te3 * qkt[:, None, :], axis=-1))
        m = jnp.max(lg, axis=-1, keepdims=True)
        p = jnp.exp(lg - m)
        att = p / jnp.sum(p, axis=-1, keepdims=True)
        a3 = att[:, :, None]
        cn = jnp.sum(se3 * a3, axis=1)
        ce = jnp.sum(ef3 * a3, axis=1)
        ct = jnp.sum(te3 * a3, axis=1)
        ctx = (dot(cn, wvn_ref[:, hs]) + dot(ce, wve_ref[:, hs])
               + dot(ct, wvt_ref[:, hs]) + bv_ref[:, hs])
        ctxs.append(ctx)

    hb = (dot(ctxs[0], wo_ref[0:128, :]) + dot(ctxs[1], wo_ref[128:256, :])
          + bo_ref[...])
    x1 = dot(em, w1_ref[0:128, :]) + dot(hb, w1_ref[128:384, :]) + b1_ref[...]
    h1 = jnp.maximum(x1, 0.0)
    out_ref[...] = dot(h1, w2_ref[...]) + b2_ref[...]


def _tc_main(se, ef2, ts2, lu2, emb, tw2, tb2, wqnT, wqtT, bq2,
             wkn, wke, wkt, wvnT, wveT, wvtT, bv2,
             woT, bo2, w1T, b12, w2T, b22, interpret=False):
    full = lambda s: pl.BlockSpec(s, lambda i: (0, 0))
    grid = _N // _B
    return pl.pallas_call(
        _tc_body,
        grid=(grid,),
        in_specs=[
            pl.BlockSpec((_BK, _DN), lambda i: (i, 0)),
            pl.BlockSpec((_BK, _DE), lambda i: (i, 0)),
            pl.BlockSpec((1, _BK // 128, 128), lambda i: (i, 0, 0)),
            pl.BlockSpec((1, _BK // 128, 128), lambda i: (i, 0, 0)),
            pl.BlockSpec((_B, _DN), lambda i: (i, 0)),
            full((1, _DT)), full((1, _DT)),
            full((_DN, _QD)), full((_DT, _QD)), full((1, _QD)),
            full((_QD, _DN)), full((_QD, _DE)), full((_QD, _DT)),
            full((_DN, _QD)), full((_DE, _QD)), full((_DT, _QD)),
            full((1, _QD)),
            full((_QD, _QD)), full((1, _QD)),
            full((_QD + _DN, _DN)), full((1, _DN)),
            full((_DN, _DN)), full((1, _DN)),
        ],
        out_specs=pl.BlockSpec((_B, _DN), lambda i: (i, 0)),
        out_shape=jax.ShapeDtypeStruct((_N, _DN), jnp.float32),
        interpret=interpret,
    )(se, ef2, ts2, lu2, emb, tw2, tb2, wqnT, wqtT, bq2,
      wkn, wke, wkt, wvnT, wveT, wvtT, bv2, woT, bo2, w1T, b12, w2T, b22)


def kernel(emb, edge_feat, timestamp, last_update, src_idx, time_w, time_b,
           Wq, bq, Wk, bk, Wv, bv, Wo, bo, W1, b1, W2, b2):
    src_emb, lu_src = _sc_gather(emb, last_update, src_idx.astype(jnp.int32))

    ts2 = timestamp.reshape(_N // _B, _BK // 128, 128)
    lu2 = lu_src.reshape(_N // _B, _BK // 128, 128)
    tw2 = time_w.reshape(1, _DT)
    tb2 = time_b.reshape(1, _DT)
    wqnT = Wq[:, :_DN].T
    wqtT = Wq[:, _DN:].T
    wkn = Wk[:, :_DN]
    wke = Wk[:, _DN:_DN + _DE]
    wkt = Wk[:, _DN + _DE:]
    wvnT = Wv[:, :_DN].T
    wveT = Wv[:, _DN:_DN + _DE].T
    wvtT = Wv[:, _DN + _DE:].T
    return _tc_main(src_emb, edge_feat, ts2, lu2, emb, tw2, tb2,
                    wqnT, wqtT, bq.reshape(1, -1),
                    wkn, wke, wkt, wvnT, wveT, wvtT, bv.reshape(1, -1),
                    Wo.T, bo.reshape(1, -1), W1.T, b1.reshape(1, -1),
                    W2.T, b2.reshape(1, -1))


# speedup vs baseline: 6.0602x; 1.0372x over previous
import functools
import math

import jax
import jax.numpy as jnp
from jax import lax
from jax.experimental import pallas as pl
from jax.experimental.pallas import tpu as pltpu
from jax.experimental.pallas import tpu_sc as plsc

_N = 10000
_K = 32
_E = _N * _K
_DN = 128
_DE = 16
_DT = 128
_QD = 256
_H = 2

_P = 2
_EP = _E // _P
_NP = _N // _P

_GRP = 40
_GPI = 5
_CH = _GRP * _GPI

_B = 200
_BK = _B * _K


def _sc_gather(emb, last_update, idx1d):
    info = plsc.get_sparse_core_info()
    nw = info.num_cores * info.num_subcores
    per_w = _EP // nw
    n_iter = per_w // _CH
    mesh = plsc.VectorSubcoreMesh(core_axis_name="c", subcore_axis_name="s")

    @functools.partial(
        pl.kernel,
        out_type=(
            jax.ShapeDtypeStruct((_EP, _DN), jnp.float32),
            jax.ShapeDtypeStruct((_EP,), jnp.float32),
        ),
        mesh=mesh,
        scratch_types=[
            pltpu.VMEM((_CH,), jnp.int32),
            pltpu.VMEM((_CH, _DN), jnp.float32),
            pltpu.VMEM((_CH,), jnp.float32),
            pltpu.SemaphoreType.DMA,
            pltpu.SemaphoreType.DMA,
        ],
    )
    def k(emb_hbm, lu_hbm, idx_hbm, se_out, lu_out, idx_v, rows_v, lus_v,
          sem_r, sem_l):
        wid = lax.axis_index("s") * info.num_cores + lax.axis_index("c")
        ebase = wid * per_w

        def body(t, _):
            e0 = ebase + t * _CH
            pltpu.sync_copy(idx_hbm.at[pl.ds(e0, _CH)], idx_v)
            cps = []
            for j in range(_GPI):
                gs = pl.ds(j * _GRP, _GRP)
                cps.append(pltpu.async_copy(
                    emb_hbm.at[idx_v.at[gs]], rows_v.at[gs], sem_r))
                cps.append(pltpu.async_copy(
                    lu_hbm.at[idx_v.at[gs]], lus_v.at[gs], sem_l))
            for cp in cps:
                cp.wait()
            pltpu.sync_copy(rows_v, se_out.at[pl.ds(e0, _CH)])
            pltpu.sync_copy(lus_v, lu_out.at[pl.ds(e0, _CH)])
            return 0

        lax.fori_loop(0, n_iter, body, 0)

    return k(emb, last_update, idx1d)


_INV2PI = 0.15915494309189535
_COSC = (0.999999210881232, -19.738980362141934, 64.92865752951663,
         -85.2716222117908, 58.790493572670144, -21.071105911444665)


def _fast_cos(x):
    u = x * _INV2PI
    u = u - jnp.round(u)
    t = u * u
    r = jnp.float32(_COSC[5])
    for c in _COSC[4::-1]:
        r = r * t + jnp.float32(c)
    return r


def _tc_body(se_ref, ef_ref, ts_ref, lu_ref, em_ref,
             tw_ref, tb_ref, wqn_ref, wqt_ref, bq_ref,
             wkn_ref, wke_ref, wkt_ref,
             wvn_ref, wve_ref, wvt_ref, bv_ref,
             wo_ref, bo_ref, w1_ref, b1_ref, w2_ref, b2_ref,
             out_ref):
    f32 = jnp.float32
    dot = functools.partial(jnp.dot, preferred_element_type=f32)
    se = se_ref[...]
    ef = ef_ref[...]
    dt = (ts_ref[...] - lu_ref[...]).reshape(_BK // 128, 128)
    tw3 = tw_ref[...].reshape(1, 1, _DT)
    tb3 = tb_ref[...].reshape(1, 1, _DT)
    dt3 = dt[:, :, None]
    te = _fast_cos(dt3 * tw3 + tb3).reshape(_BK, _DT)
    tb = tb_ref[...]
    em = em_ref[...]

    te0 = jnp.cos(tb)
    cq = dot(te0, wqt_ref[...]) + bq_ref[...]
    q = (dot(em, wqn_ref[...]) + cq) * (1.0 / math.sqrt(128.0))

    se3 = se.reshape(_B, _K, _DN)
    ef3 = ef.reshape(_B, _K, _DE)
    te3 = te.reshape(_B, _K, _DT)

    ctxs = []
    for h in range(_H):
        hs = slice(h * 128, (h + 1) * 128)
        qh = q[:, hs]
        qkn = dot(qh, wkn_ref[hs, :])
        qke = dot(qh, wke_ref[hs, :])
        qkt = dot(qh, wkt_ref[hs, :])
        prod = se3 * qkn[:, None, :] + te3 * qkt[:, None, :]
        lg = (jnp.sum(prod, axis=-1)
              + jnp.sum(ef3 * qke[:, None, :], axis=-1))
        m = jnp.max(lg, axis=-1, keepdims=True)
        p = jnp.exp(lg - m)
        att = p / jnp.sum(p, axis=-1, keepdims=True)
        a3 = att[:, :, None]
        cn = jnp.sum(se3 * a3, axis=1)
        ce = jnp.sum(ef3 * a3, axis=1)
        ct = jnp.sum(te3 * a3, axis=1)
        ctx = (dot(cn, wvn_ref[:, hs]) + dot(ce, wve_ref[:, hs])
               + dot(ct, wvt_ref[:, hs]) + bv_ref[:, hs])
        ctxs.append(ctx)

    hb = (dot(ctxs[0], wo_ref[0:128, :]) + dot(ctxs[1], wo_ref[128:256, :])
          + bo_ref[...])
    x1 = dot(em, w1_ref[0:128, :]) + dot(hb, w1_ref[128:384, :]) + b1_ref[...]
    h1 = jnp.maximum(x1, 0.0)
    out_ref[...] = dot(h1, w2_ref[...]) + b2_ref[...]


def _tc_main(se, ef2, ts2, lu2, emb, tw2, tb2, wqnT, wqtT, bq2,
             wkn, wke, wkt, wvnT, wveT, wvtT, bv2,
             woT, bo2, w1T, b12, w2T, b22, interpret=False):
    full = lambda s: pl.BlockSpec(s, lambda i: (0, 0))
    grid = _NP // _B
    return pl.pallas_call(
        _tc_body,
        grid=(grid,),
        in_specs=[
            pl.BlockSpec((_BK, _DN), lambda i: (i, 0)),
            pl.BlockSpec((_BK, _DE), lambda i: (i, 0)),
            pl.BlockSpec((1, _BK // 128, 128), lambda i: (i, 0, 0)),
            pl.BlockSpec((1, _BK // 128, 128), lambda i: (i, 0, 0)),
            pl.BlockSpec((_B, _DN), lambda i: (i, 0)),
            full((1, _DT)), full((1, _DT)),
            full((_DN, _QD)), full((_DT, _QD)), full((1, _QD)),
            full((_QD, _DN)), full((_QD, _DE)), full((_QD, _DT)),
            full((_DN, _QD)), full((_DE, _QD)), full((_DT, _QD)),
            full((1, _QD)),
            full((_QD, _QD)), full((1, _QD)),
            full((_QD + _DN, _DN)), full((1, _DN)),
            full((_DN, _DN)), full((1, _DN)),
        ],
        out_specs=pl.BlockSpec((_B, _DN), lambda i: (i, 0)),
        out_shape=jax.ShapeDtypeStruct((_NP, _DN), jnp.float32),
        interpret=interpret,
    )(se, ef2, ts2, lu2, emb, tw2, tb2, wqnT, wqtT, bq2,
      wkn, wke, wkt, wvnT, wveT, wvtT, bv2, woT, bo2, w1T, b12, w2T, b22)


def kernel(emb, edge_feat, timestamp, last_update, src_idx, time_w, time_b,
           Wq, bq, Wk, bk, Wv, bv, Wo, bo, W1, b1, W2, b2):
    idx = src_idx.astype(jnp.int32)
    tw2 = time_w.reshape(1, _DT)
    tb2 = time_b.reshape(1, _DT)
    wqnT = Wq[:, :_DN].T
    wqtT = Wq[:, _DN:].T
    wkn = Wk[:, :_DN]
    wke = Wk[:, _DN:_DN + _DE]
    wkt = Wk[:, _DN + _DE:]
    wvnT = Wv[:, :_DN].T
    wveT = Wv[:, _DN:_DN + _DE].T
    wvtT = Wv[:, _DN + _DE:].T
    bq2 = bq.reshape(1, -1)
    bv2 = bv.reshape(1, -1)
    woT, bo2 = Wo.T, bo.reshape(1, -1)
    w1T, b12 = W1.T, b1.reshape(1, -1)
    w2T, b22 = W2.T, b2.reshape(1, -1)

    outs = []
    for p in range(_P):
        es = slice(p * _EP, (p + 1) * _EP)
        ns = slice(p * _NP, (p + 1) * _NP)
        src_emb, lu_src = _sc_gather(emb, last_update, idx[es])
        ts2 = timestamp[es].reshape(_NP // _B, _BK // 128, 128)
        lu2 = lu_src.reshape(_NP // _B, _BK // 128, 128)
        outs.append(_tc_main(src_emb, edge_feat[es], ts2, lu2, emb[ns],
                             tw2, tb2, wqnT, wqtT, bq2,
                             wkn, wke, wkt, wvnT, wveT, wvtT, bv2,
                             woT, bo2, w1T, b12, w2T, b22))
    return jnp.concatenate(outs, axis=0)


# speedup vs baseline: 6.1235x; 1.0104x over previous
import functools
import math

import jax
import jax.numpy as jnp
from jax import lax
from jax.experimental import pallas as pl
from jax.experimental.pallas import tpu as pltpu
from jax.experimental.pallas import tpu_sc as plsc

_N = 10000
_K = 32
_E = _N * _K
_DN = 128
_DE = 16
_DT = 128
_QD = 256
_H = 2

_P = 2
_EP = _E // _P
_NP = _N // _P

_GRP = 40
_GPI = 5
_CH = _GRP * _GPI

_B = 200
_BK = _B * _K


def _sc_gather(emb, last_update, idx1d):
    info = plsc.get_sparse_core_info()
    nw = info.num_cores * info.num_subcores
    per_w = _EP // nw
    n_iter = per_w // _CH
    nbuf = 3
    mesh = plsc.VectorSubcoreMesh(core_axis_name="c", subcore_axis_name="s")

    @functools.partial(
        pl.kernel,
        out_type=(
            jax.ShapeDtypeStruct((_EP, _DN), jnp.float32),
            jax.ShapeDtypeStruct((_EP,), jnp.float32),
        ),
        mesh=mesh,
        scratch_types=[
            pltpu.VMEM((per_w,), jnp.int32),
            pltpu.VMEM((nbuf, _CH, _DN), jnp.float32),
            pltpu.VMEM((nbuf * _CH,), jnp.float32),
            pltpu.SemaphoreType.DMA,
            pltpu.SemaphoreType.DMA,
            pltpu.SemaphoreType.DMA,
            pltpu.SemaphoreType.DMA,
        ],
    )
    def k(emb_hbm, lu_hbm, idx_hbm, se_out, lu_out, idx_v, rows_v, lus_v,
          sem_r, sem_l, sem_wr, sem_wl):
        wid = lax.axis_index("s") * info.num_cores + lax.axis_index("c")
        ebase = wid * per_w
        pltpu.sync_copy(idx_hbm.at[pl.ds(ebase, per_w)], idx_v)

        def g_copies(t, ph, make):
            mk = pltpu.make_async_copy if make else pltpu.async_copy
            cps = []
            lb = pl.multiple_of(ph * _CH, 8)
            for j in range(_GPI):
                s0 = t * _CH + j * _GRP
                cps.append(mk(emb_hbm.at[idx_v.at[pl.ds(s0, _GRP)]],
                              rows_v.at[ph, pl.ds(j * _GRP, _GRP)], sem_r))
                cps.append(mk(lu_hbm.at[idx_v.at[pl.ds(s0, _GRP)]],
                              lus_v.at[pl.ds(lb + j * _GRP, _GRP)], sem_l))
            return cps

        def wb_copies(t, ph, make):
            mk = pltpu.make_async_copy if make else pltpu.async_copy
            e0 = ebase + t * _CH
            lb = pl.multiple_of(ph * _CH, 8)
            return [mk(rows_v.at[ph], se_out.at[pl.ds(e0, _CH)], sem_wr),
                    mk(lus_v.at[pl.ds(lb, _CH)], lu_out.at[pl.ds(e0, _CH)],
                       sem_wl)]

        fire_g = lambda t, ph: g_copies(t, ph, False)
        wait_g = lambda t, ph: [c.wait() for c in g_copies(t, ph, True)]
        fire_wb = lambda t, ph: wb_copies(t, ph, False)
        wait_wb = lambda t, ph: [c.wait() for c in wb_copies(t, ph, True)]

        fire_g(0, 0)
        fire_g(1, 1)
        wait_g(0, 0)
        fire_wb(0, 0)
        fire_g(2, 2)

        def body(t, _):
            ph = lax.rem(t, nbuf)
            wait_g(t, ph)
            fire_wb(t, ph)
            wait_wb(t - 1, lax.rem(t - 1, nbuf))
            fire_g(t + 2, lax.rem(t + 2, nbuf))
            return 0

        lax.fori_loop(1, n_iter - 2, body, 0)

        t = n_iter - 2
        wait_g(t, lax.rem(t, nbuf))
        fire_wb(t, lax.rem(t, nbuf))
        wait_wb(t - 1, lax.rem(t - 1, nbuf))
        t = n_iter - 1
        wait_g(t, lax.rem(t, nbuf))
        fire_wb(t, lax.rem(t, nbuf))
        wait_wb(t - 1, lax.rem(t - 1, nbuf))
        wait_wb(t, lax.rem(t, nbuf))

    return k(emb, last_update, idx1d)


_INV2PI = 0.15915494309189535
_COSC = (0.999999210881232, -19.738980362141934, 64.92865752951663,
         -85.2716222117908, 58.790493572670144, -21.071105911444665)


def _fast_cos(x):
    u = x * _INV2PI
    u = u - jnp.round(u)
    t = u * u
    r = jnp.float32(_COSC[5])
    for c in _COSC[4::-1]:
        r = r * t + jnp.float32(c)
    return r


def _tc_body(se_ref, ef_ref, ts_ref, lu_ref, em_ref,
             tw_ref, tb_ref, wqn_ref, wqt_ref, bq_ref,
             wkn_ref, wke_ref, wkt_ref,
             wvn_ref, wve_ref, wvt_ref, bv_ref,
             wo_ref, bo_ref, w1_ref, b1_ref, w2_ref, b2_ref,
             out_ref):
    f32 = jnp.float32
    dot = functools.partial(jnp.dot, preferred_element_type=f32)
    se = se_ref[...]
    ef = ef_ref[...]
    dt = (ts_ref[...] - lu_ref[...]).reshape(_BK // 128, 128)
    tw3 = tw_ref[...].reshape(1, 1, _DT)
    tb3 = tb_ref[...].reshape(1, 1, _DT)
    dt3 = dt[:, :, None]
    te = _fast_cos(dt3 * tw3 + tb3).reshape(_BK, _DT)
    tb = tb_ref[...]
    em = em_ref[...]

    te0 = jnp.cos(tb)
    cq = dot(te0, wqt_ref[...]) + bq_ref[...]
    q = (dot(em, wqn_ref[...]) + cq) * (1.0 / math.sqrt(128.0))

    se3 = se.reshape(_B, _K, _DN)
    ef3 = ef.reshape(_B, _K, _DE)
    te3 = te.reshape(_B, _K, _DT)

    ctxs = []
    for h in range(_H):
        hs = slice(h * 128, (h + 1) * 128)
        qh = q[:, hs]
        qkn = dot(qh, wkn_ref[hs, :])
        qke = dot(qh, wke_ref[hs, :])
        qkt = dot(qh, wkt_ref[hs, :])
        prod = se3 * qkn[:, None, :] + te3 * qkt[:, None, :]
        lg = (jnp.sum(prod, axis=-1)
              + jnp.sum(ef3 * qke[:, None, :], axis=-1))
        m = jnp.max(lg, axis=-1, keepdims=True)
        p = jnp.exp(lg - m)
        att = p / jnp.sum(p, axis=-1, keepdims=True)
        a3 = att[:, :, None]
        cn = jnp.sum(se3 * a3, axis=1)
        ce = jnp.sum(ef3 * a3, axis=1)
        ct = jnp.sum(te3 * a3, axis=1)
        ctx = (dot(cn, wvn_ref[:, hs]) + dot(ce, wve_ref[:, hs])
               + dot(ct, wvt_ref[:, hs]) + bv_ref[:, hs])
        ctxs.append(ctx)

    hb = (dot(ctxs[0], wo_ref[0:128, :]) + dot(ctxs[1], wo_ref[128:256, :])
          + bo_ref[...])
    x1 = dot(em, w1_ref[0:128, :]) + dot(hb, w1_ref[128:384, :]) + b1_ref[...]
    h1 = jnp.maximum(x1, 0.0)
    out_ref[...] = dot(h1, w2_ref[...]) + b2_ref[...]


def _tc_main(se, ef2, ts2, lu2, emb, tw2, tb2, wqnT, wqtT, bq2,
             wkn, wke, wkt, wvnT, wveT, wvtT, bv2,
             woT, bo2, w1T, b12, w2T, b22, interpret=False):
    full = lambda s: pl.BlockSpec(s, lambda i: (0, 0))
    grid = _NP // _B
    return pl.pallas_call(
        _tc_body,
        grid=(grid,),
        in_specs=[
            pl.BlockSpec((_BK, _DN), lambda i: (i, 0)),
            pl.BlockSpec((_BK, _DE), lambda i: (i, 0)),
            pl.BlockSpec((1, _BK // 128, 128), lambda i: (i, 0, 0)),
            pl.BlockSpec((1, _BK // 128, 128), lambda i: (i, 0, 0)),
            pl.BlockSpec((_B, _DN), lambda i: (i, 0)),
            full((1, _DT)), full((1, _DT)),
            full((_DN, _QD)), full((_DT, _QD)), full((1, _QD)),
            full((_QD, _DN)), full((_QD, _DE)), full((_QD, _DT)),
            full((_DN, _QD)), full((_DE, _QD)), full((_DT, _QD)),
            full((1, _QD)),
            full((_QD, _QD)), full((1, _QD)),
            full((_QD + _DN, _DN)), full((1, _DN)),
            full((_DN, _DN)), full((1, _DN)),
        ],
        out_specs=pl.BlockSpec((_B, _DN), lambda i: (i, 0)),
        out_shape=jax.ShapeDtypeStruct((_NP, _DN), jnp.float32),
        interpret=interpret,
    )(se, ef2, ts2, lu2, emb, tw2, tb2, wqnT, wqtT, bq2,
      wkn, wke, wkt, wvnT, wveT, wvtT, bv2, woT, bo2, w1T, b12, w2T, b22)


def kernel(emb, edge_feat, timestamp, last_update, src_idx, time_w, time_b,
           Wq, bq, Wk, bk, Wv, bv, Wo, bo, W1, b1, W2, b2):
    idx = src_idx.astype(jnp.int32)
    tw2 = time_w.reshape(1, _DT)
    tb2 = time_b.reshape(1, _DT)
    wqnT = Wq[:, :_DN].T
    wqtT = Wq[:, _DN:].T
    wkn = Wk[:, :_DN]
    wke = Wk[:, _DN:_DN + _DE]
    wkt = Wk[:, _DN + _DE:]
    wvnT = Wv[:, :_DN].T
    wveT = Wv[:, _DN:_DN + _DE].T
    wvtT = Wv[:, _DN + _DE:].T
    bq2 = bq.reshape(1, -1)
    bv2 = bv.reshape(1, -1)
    woT, bo2 = Wo.T, bo.reshape(1, -1)
    w1T, b12 = W1.T, b1.reshape(1, -1)
    w2T, b22 = W2.T, b2.reshape(1, -1)

    outs = []
    for p in range(_P):
        es = slice(p * _EP, (p + 1) * _EP)
        ns = slice(p * _NP, (p + 1) * _NP)
        src_emb, lu_src = _sc_gather(emb, last_update, idx[es])
        ts2 = timestamp[es].reshape(_NP // _B, _BK // 128, 128)
        lu2 = lu_src.reshape(_NP // _B, _BK // 128, 128)
        outs.append(_tc_main(src_emb, edge_feat[es], ts2, lu2, emb[ns],
                             tw2, tb2, wqnT, wqtT, bq2,
                             wkn, wke, wkt, wvnT, wveT, wvtT, bv2,
                             woT, bo2, w1T, b12, w2T, b22))
    return jnp.concatenate(outs, axis=0)


# speedup vs baseline: 6.2719x; 1.0242x over previous
import functools
import math

import jax
import jax.numpy as jnp
from jax import lax
from jax.experimental import pallas as pl
from jax.experimental.pallas import tpu as pltpu
from jax.experimental.pallas import tpu_sc as plsc

_N = 10000
_K = 32
_E = _N * _K
_DN = 128
_DE = 16
_DT = 128
_QD = 256
_H = 2

_P = 2
_EP = _E // _P
_NP = _N // _P

_GRP = 40
_GPI = 5
_CH = _GRP * _GPI

_B = 200
_BK = _B * _K


def _sc_gather(emb, last_update, idx1d):
    info = plsc.get_sparse_core_info()
    nw = info.num_cores * info.num_subcores
    per_w = _EP // nw
    n_iter = per_w // _CH
    nbuf = 3
    mesh = plsc.VectorSubcoreMesh(core_axis_name="c", subcore_axis_name="s")

    @functools.partial(
        pl.kernel,
        out_type=(
            jax.ShapeDtypeStruct((_EP, _DN), jnp.float32),
            jax.ShapeDtypeStruct((_EP,), jnp.float32),
        ),
        mesh=mesh,
        scratch_types=[
            pltpu.VMEM((per_w,), jnp.int32),
            pltpu.VMEM((nbuf, _CH, _DN), jnp.float32),
            pltpu.VMEM((nbuf * _CH,), jnp.float32),
            pltpu.SemaphoreType.DMA,
            pltpu.SemaphoreType.DMA,
            pltpu.SemaphoreType.DMA,
            pltpu.SemaphoreType.DMA,
        ],
    )
    def k(emb_hbm, lu_hbm, idx_hbm, se_out, lu_out, idx_v, rows_v, lus_v,
          sem_r, sem_l, sem_wr, sem_wl):
        wid = lax.axis_index("s") * info.num_cores + lax.axis_index("c")
        ebase = wid * per_w
        pltpu.sync_copy(idx_hbm.at[pl.ds(ebase, per_w)], idx_v)

        def g_copies(t, ph, make):
            mk = pltpu.make_async_copy if make else pltpu.async_copy
            cps = []
            lb = pl.multiple_of(ph * _CH, 8)
            for j in range(_GPI):
                s0 = t * _CH + j * _GRP
                cps.append(mk(emb_hbm.at[idx_v.at[pl.ds(s0, _GRP)]],
                              rows_v.at[ph, pl.ds(j * _GRP, _GRP)], sem_r))
                cps.append(mk(lu_hbm.at[idx_v.at[pl.ds(s0, _GRP)]],
                              lus_v.at[pl.ds(lb + j * _GRP, _GRP)], sem_l))
            return cps

        def wb_copies(t, ph, make):
            mk = pltpu.make_async_copy if make else pltpu.async_copy
            e0 = ebase + t * _CH
            lb = pl.multiple_of(ph * _CH, 8)
            return [mk(rows_v.at[ph], se_out.at[pl.ds(e0, _CH)], sem_wr),
                    mk(lus_v.at[pl.ds(lb, _CH)], lu_out.at[pl.ds(e0, _CH)],
                       sem_wl)]

        fire_g = lambda t, ph: g_copies(t, ph, False)
        wait_g = lambda t, ph: [c.wait() for c in g_copies(t, ph, True)]
        fire_wb = lambda t, ph: wb_copies(t, ph, False)
        wait_wb = lambda t, ph: [c.wait() for c in wb_copies(t, ph, True)]

        fire_g(0, 0)
        fire_g(1, 1)
        wait_g(0, 0)
        fire_wb(0, 0)
        fire_g(2, 2)

        def body(t, _):
            ph = lax.rem(t, nbuf)
            wait_g(t, ph)
            fire_wb(t, ph)
            wait_wb(t - 1, lax.rem(t - 1, nbuf))
            fire_g(t + 2, lax.rem(t + 2, nbuf))
            return 0

        lax.fori_loop(1, n_iter - 2, body, 0)

        t = n_iter - 2
        wait_g(t, lax.rem(t, nbuf))
        fire_wb(t, lax.rem(t, nbuf))
        wait_wb(t - 1, lax.rem(t - 1, nbuf))
        t = n_iter - 1
        wait_g(t, lax.rem(t, nbuf))
        fire_wb(t, lax.rem(t, nbuf))
        wait_wb(t - 1, lax.rem(t - 1, nbuf))
        wait_wb(t, lax.rem(t, nbuf))

    return k(emb, last_update, idx1d)


_INV2PI = 0.15915494309189535
_COSC = (0.999999210881232, -19.738980362141934, 64.92865752951663,
         -85.2716222117908, 58.790493572670144, -21.071105911444665)


def _fast_cos(x):
    u = x * _INV2PI
    u = u - jnp.round(u)
    t = u * u
    r = jnp.float32(_COSC[5])
    for c in _COSC[4::-1]:
        r = r * t + jnp.float32(c)
    return r


def _tc_body(se_ref, ef_ref, ts_ref, lu_ref, em_ref,
             tw_ref, tb_ref, wqn_ref, wqt_ref, bq_ref,
             wkn_ref, wke_ref, wkt_ref,
             wvn_ref, wve_ref, wvt_ref, bv_ref,
             wo_ref, bo_ref, w1_ref, b1_ref, w2_ref, b2_ref,
             out_ref):
    f32 = jnp.float32
    dot = functools.partial(jnp.dot, preferred_element_type=f32)
    se = se_ref[...]
    ef = ef_ref[...]
    dt = (ts_ref[...] - lu_ref[...]).reshape(_BK // 128, 128)
    tw3 = tw_ref[...].reshape(1, 1, _DT)
    tb3 = tb_ref[...].reshape(1, 1, _DT)
    dt3 = dt[:, :, None]
    te = _fast_cos(dt3 * tw3 + tb3).reshape(_BK, _DT)
    tb = tb_ref[...]
    em = em_ref[...]

    te0 = jnp.cos(tb)
    cq = dot(te0, wqt_ref[...]) + bq_ref[...]
    q = (dot(em, wqn_ref[...]) + cq) * (1.0 / math.sqrt(128.0))

    se3 = se.reshape(_B, _K, _DN)
    ef3 = ef.reshape(_B, _K, _DE)
    te3 = te.reshape(_B, _K, _DT)

    ctxs = []
    for h in range(_H):
        hs = slice(h * 128, (h + 1) * 128)
        qh = q[:, hs]
        qkn = dot(qh, wkn_ref[hs, :])
        qke = dot(qh, wke_ref[hs, :])
        qkt = dot(qh, wkt_ref[hs, :])
        prod = se3 * qkn[:, None, :] + te3 * qkt[:, None, :]
        lg = (jnp.sum(prod, axis=-1)
              + jnp.sum(ef3 * qke[:, None, :], axis=-1))
        m = jnp.max(lg, axis=-1, keepdims=True)
        p = jnp.exp(lg - m)
        att = p / jnp.sum(p, axis=-1, keepdims=True)
        a3 = att[:, :, None]
        cn = jnp.sum(se3 * a3, axis=1)
        ce = jnp.sum(ef3 * a3, axis=1)
        ct = jnp.sum(te3 * a3, axis=1)
        ctx = (dot(cn, wvn_ref[:, hs]) + dot(ce, wve_ref[:, hs])
               + dot(ct, wvt_ref[:, hs]) + bv_ref[:, hs])
        ctxs.append(ctx)

    hb = (dot(ctxs[0], wo_ref[0:128, :]) + dot(ctxs[1], wo_ref[128:256, :])
          + bo_ref[...])
    x1 = dot(em, w1_ref[0:128, :]) + dot(hb, w1_ref[128:384, :]) + b1_ref[...]
    h1 = jnp.maximum(x1, 0.0)
    out_ref[...] = dot(h1, w2_ref[...]) + b2_ref[...]


def _tc_main(se, ef2, ts2, lu2, emb, tw2, tb2, wqnT, wqtT, bq2,
             wkn, wke, wkt, wvnT, wveT, wvtT, bv2,
             woT, bo2, w1T, b12, w2T, b22, part=0, interpret=False):
    full = lambda s: pl.BlockSpec(s, lambda i: (0, 0))
    grid = _NP // _B
    off = part * grid
    return pl.pallas_call(
        _tc_body,
        grid=(grid,),
        in_specs=[
            pl.BlockSpec((_BK, _DN), lambda i: (i, 0)),
            pl.BlockSpec((_BK, _DE), lambda i: (i + off, 0)),
            pl.BlockSpec((1, _BK // 128, 128), lambda i: (i + off, 0, 0)),
            pl.BlockSpec((1, _BK // 128, 128), lambda i: (i, 0, 0)),
            pl.BlockSpec((_B, _DN), lambda i: (i + off, 0)),
            full((1, _DT)), full((1, _DT)),
            full((_DN, _QD)), full((_DT, _QD)), full((1, _QD)),
            full((_QD, _DN)), full((_QD, _DE)), full((_QD, _DT)),
            full((_DN, _QD)), full((_DE, _QD)), full((_DT, _QD)),
            full((1, _QD)),
            full((_QD, _QD)), full((1, _QD)),
            full((_QD + _DN, _DN)), full((1, _DN)),
            full((_DN, _DN)), full((1, _DN)),
        ],
        out_specs=pl.BlockSpec((_B, _DN), lambda i: (i, 0)),
        out_shape=jax.ShapeDtypeStruct((_NP, _DN), jnp.float32),
        interpret=interpret,
    )(se, ef2, ts2, lu2, emb, tw2, tb2, wqnT, wqtT, bq2,
      wkn, wke, wkt, wvnT, wveT, wvtT, bv2, woT, bo2, w1T, b12, w2T, b22)


def kernel(emb, edge_feat, timestamp, last_update, src_idx, time_w, time_b,
           Wq, bq, Wk, bk, Wv, bv, Wo, bo, W1, b1, W2, b2):
    idx = src_idx.astype(jnp.int32)
    tw2 = time_w.reshape(1, _DT)
    tb2 = time_b.reshape(1, _DT)
    wqnT = Wq[:, :_DN].T
    wqtT = Wq[:, _DN:].T
    wkn = Wk[:, :_DN]
    wke = Wk[:, _DN:_DN + _DE]
    wkt = Wk[:, _DN + _DE:]
    wvnT = Wv[:, :_DN].T
    wveT = Wv[:, _DN:_DN + _DE].T
    wvtT = Wv[:, _DN + _DE:].T
    bq2 = bq.reshape(1, -1)
    bv2 = bv.reshape(1, -1)
    woT, bo2 = Wo.T, bo.reshape(1, -1)
    w1T, b12 = W1.T, b1.reshape(1, -1)
    w2T, b22 = W2.T, b2.reshape(1, -1)

    ts2 = timestamp.reshape(_N // _B, _BK // 128, 128)
    outs = []
    for p in range(_P):
        es = slice(p * _EP, (p + 1) * _EP)
        src_emb, lu_src = _sc_gather(emb, last_update, idx[es])
        lu2 = lu_src.reshape(_NP // _B, _BK // 128, 128)
        outs.append(_tc_main(src_emb, edge_feat, ts2, lu2, emb,
                             tw2, tb2, wqnT, wqtT, bq2,
                             wkn, wke, wkt, wvnT, wveT, wvtT, bv2,
                             woT, bo2, w1T, b12, w2T, b22, part=p))
    return jnp.concatenate(outs, axis=0)


# speedup vs baseline: 7.1801x; 1.1448x over previous
import functools
import math

import jax
import jax.numpy as jnp
from jax import lax
from jax.experimental import pallas as pl
from jax.experimental.pallas import tpu as pltpu
from jax.experimental.pallas import tpu_sc as plsc

_N = 10000
_K = 32
_E = _N * _K
_DN = 128
_DE = 16
_DT = 128
_QD = 256
_H = 2

_P = 2
_EP = _E // _P
_NP = _N // _P

_GRP = 40
_GPI = 5
_CH = _GRP * _GPI

_B = 200
_BK = _B * _K


def _sc_gather(emb, last_update, idx1d):
    info = plsc.get_sparse_core_info()
    nw = info.num_cores * info.num_subcores
    per_w = _EP // nw
    n_iter = per_w // _CH
    nbuf = 3
    mesh = plsc.VectorSubcoreMesh(core_axis_name="c", subcore_axis_name="s")

    @functools.partial(
        pl.kernel,
        out_type=(
            jax.ShapeDtypeStruct((_EP, _DN), jnp.float32),
            jax.ShapeDtypeStruct((_EP,), jnp.float32),
        ),
        mesh=mesh,
        scratch_types=[
            pltpu.VMEM((per_w,), jnp.int32),
            pltpu.VMEM((nbuf, _CH, _DN), jnp.float32),
            pltpu.VMEM((nbuf * _CH,), jnp.float32),
            pltpu.SemaphoreType.DMA,
            pltpu.SemaphoreType.DMA,
            pltpu.SemaphoreType.DMA,
            pltpu.SemaphoreType.DMA,
        ],
    )
    def k(emb_hbm, lu_hbm, idx_hbm, se_out, lu_out, idx_v, rows_v, lus_v,
          sem_r, sem_l, sem_wr, sem_wl):
        wid = lax.axis_index("s") * info.num_cores + lax.axis_index("c")
        ebase = wid * per_w
        pltpu.sync_copy(idx_hbm.at[pl.ds(ebase, per_w)], idx_v)

        def g_copies(t, ph, make):
            mk = pltpu.make_async_copy if make else pltpu.async_copy
            cps = []
            lb = pl.multiple_of(ph * _CH, 8)
            for j in range(_GPI):
                s0 = t * _CH + j * _GRP
                cps.append(mk(emb_hbm.at[idx_v.at[pl.ds(s0, _GRP)]],
                              rows_v.at[ph, pl.ds(j * _GRP, _GRP)], sem_r))
                cps.append(mk(lu_hbm.at[idx_v.at[pl.ds(s0, _GRP)]],
                              lus_v.at[pl.ds(lb + j * _GRP, _GRP)], sem_l))
            return cps

        def wb_copies(t, ph, make):
            mk = pltpu.make_async_copy if make else pltpu.async_copy
            e0 = ebase + t * _CH
            lb = pl.multiple_of(ph * _CH, 8)
            return [mk(rows_v.at[ph], se_out.at[pl.ds(e0, _CH)], sem_wr),
                    mk(lus_v.at[pl.ds(lb, _CH)], lu_out.at[pl.ds(e0, _CH)],
                       sem_wl)]

        fire_g = lambda t, ph: g_copies(t, ph, False)
        wait_g = lambda t, ph: [c.wait() for c in g_copies(t, ph, True)]
        fire_wb = lambda t, ph: wb_copies(t, ph, False)
        wait_wb = lambda t, ph: [c.wait() for c in wb_copies(t, ph, True)]

        fire_g(0, 0)
        fire_g(1, 1)
        wait_g(0, 0)
        fire_wb(0, 0)
        fire_g(2, 2)

        def body(t, _):
            ph = lax.rem(t, nbuf)
            wait_g(t, ph)
            fire_wb(t, ph)
            wait_wb(t - 1, lax.rem(t - 1, nbuf))
            fire_g(t + 2, lax.rem(t + 2, nbuf))
            return 0

        lax.fori_loop(1, n_iter - 2, body, 0)

        t = n_iter - 2
        wait_g(t, lax.rem(t, nbuf))
        fire_wb(t, lax.rem(t, nbuf))
        wait_wb(t - 1, lax.rem(t - 1, nbuf))
        t = n_iter - 1
        wait_g(t, lax.rem(t, nbuf))
        fire_wb(t, lax.rem(t, nbuf))
        wait_wb(t - 1, lax.rem(t - 1, nbuf))
        wait_wb(t, lax.rem(t, nbuf))

    return k(emb, last_update, idx1d)


_INV2PI = 0.15915494309189535
_COSC = (0.999999210881232, -19.738980362141934, 64.92865752951663,
         -85.2716222117908, 58.790493572670144, -21.071105911444665)


def _fast_cos(x):
    u = x * _INV2PI
    u = u - jnp.round(u)
    t = u * u
    r = jnp.float32(_COSC[5])
    for c in _COSC[4::-1]:
        r = r * t + jnp.float32(c)
    return r


def _tc_body(se_ref, ef_ref, ts_ref, lu_ref, em_ref,
             tw_ref, tb_ref, wqn_ref, wqt_ref, bq_ref,
             wkn_ref, wke_ref, wkt_ref,
             wvn_ref, wve_ref, wvt_ref, bv_ref,
             wo_ref, bo_ref, w1_ref, b1_ref, w2_ref, b2_ref,
             out_ref):
    f32 = jnp.float32
    dot = functools.partial(jnp.dot, preferred_element_type=f32)
    se = se_ref[...]
    ef = ef_ref[...]
    dt = (ts_ref[...] - lu_ref[...]).reshape(_BK // 128, 128)
    tw3 = tw_ref[...].reshape(1, 1, _DT)
    tb3 = tb_ref[...].reshape(1, 1, _DT)
    dt3 = dt[:, :, None]
    te = _fast_cos(dt3 * tw3 + tb3).reshape(_BK, _DT)
    tb = tb_ref[...]
    em = em_ref[...]

    te0 = jnp.cos(tb)
    cq = dot(te0, wqt_ref[...]) + bq_ref[...]
    q = (dot(em, wqn_ref[...]) + cq) * (1.0 / math.sqrt(128.0))

    se3 = se.reshape(_B, _K, _DN)
    ef3 = ef.reshape(_B, _K, _DE)
    te3 = te.reshape(_B, _K, _DT)

    ctxs = []
    for h in range(_H):
        hs = slice(h * 128, (h + 1) * 128)
        qh = q[:, hs]
        qkn = dot(qh, wkn_ref[hs, :])
        qke = dot(qh, wke_ref[hs, :])
        qkt = dot(qh, wkt_ref[hs, :])
        prod = se3 * qkn[:, None, :] + te3 * qkt[:, None, :]
        lg3 = (jnp.sum(prod, axis=-1, keepdims=True)
               + jnp.sum(ef3 * qke[:, None, :], axis=-1,
                         keepdims=True))
        m = jnp.max(lg3, axis=1, keepdims=True)
        p = jnp.exp(lg3 - m)
        a3 = p / jnp.sum(p, axis=1, keepdims=True)
        cn = jnp.sum(se3 * a3, axis=1)
        ce = jnp.sum(ef3 * a3, axis=1)
        ct = jnp.sum(te3 * a3, axis=1)
        ctx = (dot(cn, wvn_ref[:, hs]) + dot(ce, wve_ref[:, hs])
               + dot(ct, wvt_ref[:, hs]) + bv_ref[:, hs])
        ctxs.append(ctx)

    hb = (dot(ctxs[0], wo_ref[0:128, :]) + dot(ctxs[1], wo_ref[128:256, :])
          + bo_ref[...])
    x1 = dot(em, w1_ref[0:128, :]) + dot(hb, w1_ref[128:384, :]) + b1_ref[...]
    h1 = jnp.maximum(x1, 0.0)
    out_ref[...] = dot(h1, w2_ref[...]) + b2_ref[...]


def _tc_main(se, ef2, ts2, lu2, emb, tw2, tb2, wqnT, wqtT, bq2,
             wkn, wke, wkt, wvnT, wveT, wvtT, bv2,
             woT, bo2, w1T, b12, w2T, b22, part=0, interpret=False):
    full = lambda s: pl.BlockSpec(s, lambda i: (0, 0))
    grid = _NP // _B
    off = part * grid
    return pl.pallas_call(
        _tc_body,
        grid=(grid,),
        in_specs=[
            pl.BlockSpec((_BK, _DN), lambda i: (i, 0)),
            pl.BlockSpec((_BK, _DE), lambda i: (i + off, 0)),
            pl.BlockSpec((1, _BK // 128, 128), lambda i: (i + off, 0, 0)),
            pl.BlockSpec((1, _BK // 128, 128), lambda i: (i, 0, 0)),
            pl.BlockSpec((_B, _DN), lambda i: (i + off, 0)),
            full((1, _DT)), full((1, _DT)),
            full((_DN, _QD)), full((_DT, _QD)), full((1, _QD)),
            full((_QD, _DN)), full((_QD, _DE)), full((_QD, _DT)),
            full((_DN, _QD)), full((_DE, _QD)), full((_DT, _QD)),
            full((1, _QD)),
            full((_QD, _QD)), full((1, _QD)),
            full((_QD + _DN, _DN)), full((1, _DN)),
            full((_DN, _DN)), full((1, _DN)),
        ],
        out_specs=pl.BlockSpec((_B, _DN), lambda i: (i, 0)),
        out_shape=jax.ShapeDtypeStruct((_NP, _DN), jnp.float32),
        interpret=interpret,
    )(se, ef2, ts2, lu2, emb, tw2, tb2, wqnT, wqtT, bq2,
      wkn, wke, wkt, wvnT, wveT, wvtT, bv2, woT, bo2, w1T, b12, w2T, b22)


def kernel(emb, edge_feat, timestamp, last_update, src_idx, time_w, time_b,
           Wq, bq, Wk, bk, Wv, bv, Wo, bo, W1, b1, W2, b2):
    idx = src_idx.astype(jnp.int32)
    tw2 = time_w.reshape(1, _DT)
    tb2 = time_b.reshape(1, _DT)
    wqnT = Wq[:, :_DN].T
    wqtT = Wq[:, _DN:].T
    wkn = Wk[:, :_DN]
    wke = Wk[:, _DN:_DN + _DE]
    wkt = Wk[:, _DN + _DE:]
    wvnT = Wv[:, :_DN].T
    wveT = Wv[:, _DN:_DN + _DE].T
    wvtT = Wv[:, _DN + _DE:].T
    bq2 = bq.reshape(1, -1)
    bv2 = bv.reshape(1, -1)
    woT, bo2 = Wo.T, bo.reshape(1, -1)
    w1T, b12 = W1.T, b1.reshape(1, -1)
    w2T, b22 = W2.T, b2.reshape(1, -1)

    ts2 = timestamp.reshape(_N // _B, _BK // 128, 128)
    outs = []
    for p in range(_P):
        es = slice(p * _EP, (p + 1) * _EP)
        src_emb, lu_src = _sc_gather(emb, last_update, idx[es])
        lu2 = lu_src.reshape(_NP // _B, _BK // 128, 128)
        outs.append(_tc_main(src_emb, edge_feat, ts2, lu2, emb,
                             tw2, tb2, wqnT, wqtT, bq2,
                             wkn, wke, wkt, wvnT, wveT, wvtT, bv2,
                             woT, bo2, w1T, b12, w2T, b22, part=p))
    return jnp.concatenate(outs, axis=0)


# speedup vs baseline: 7.1992x; 1.0027x over previous
import functools
import math

import jax
import jax.numpy as jnp
from jax import lax
from jax.experimental import pallas as pl
from jax.experimental.pallas import tpu as pltpu
from jax.experimental.pallas import tpu_sc as plsc

_N = 10000
_K = 32
_E = _N * _K
_DN = 128
_DE = 16
_DT = 128
_QD = 256
_H = 2

_P = 2
_EP = _E // _P
_NP = _N // _P

_GRP = 40
_GPI = 5
_CH = _GRP * _GPI

_B = 200
_BK = _B * _K


def _sc_gather(emb, last_update, idx1d):
    info = plsc.get_sparse_core_info()
    nw = info.num_cores * info.num_subcores
    per_w = _EP // nw
    n_iter = per_w // _CH
    nbuf = 3
    mesh = plsc.VectorSubcoreMesh(core_axis_name="c", subcore_axis_name="s")

    @functools.partial(
        pl.kernel,
        out_type=(
            jax.ShapeDtypeStruct((_EP, _DN), jnp.float32),
            jax.ShapeDtypeStruct((_EP,), jnp.float32),
        ),
        mesh=mesh,
        scratch_types=[
            pltpu.VMEM((per_w,), jnp.int32),
            pltpu.VMEM((nbuf, _CH, _DN), jnp.float32),
            pltpu.VMEM((nbuf * _CH,), jnp.float32),
            pltpu.SemaphoreType.DMA,
            pltpu.SemaphoreType.DMA,
            pltpu.SemaphoreType.DMA,
            pltpu.SemaphoreType.DMA,
        ],
    )
    def k(emb_hbm, lu_hbm, idx_hbm, se_out, lu_out, idx_v, rows_v, lus_v,
          sem_r, sem_l, sem_wr, sem_wl):
        wid = lax.axis_index("s") * info.num_cores + lax.axis_index("c")
        ebase = wid * per_w
        pltpu.sync_copy(idx_hbm.at[pl.ds(ebase, per_w)], idx_v)

        def g_copies(t, ph, make):
            mk = pltpu.make_async_copy if make else pltpu.async_copy
            cps = []
            lb = pl.multiple_of(ph * _CH, 8)
            for j in range(_GPI):
                s0 = t * _CH + j * _GRP
                cps.append(mk(emb_hbm.at[idx_v.at[pl.ds(s0, _GRP)]],
                              rows_v.at[ph, pl.ds(j * _GRP, _GRP)], sem_r))
                cps.append(mk(lu_hbm.at[idx_v.at[pl.ds(s0, _GRP)]],
                              lus_v.at[pl.ds(lb + j * _GRP, _GRP)], sem_l))
            return cps

        def wb_copies(t, ph, make):
            mk = pltpu.make_async_copy if make else pltpu.async_copy
            e0 = ebase + t * _CH
            lb = pl.multiple_of(ph * _CH, 8)
            return [mk(rows_v.at[ph], se_out.at[pl.ds(e0, _CH)], sem_wr),
                    mk(lus_v.at[pl.ds(lb, _CH)], lu_out.at[pl.ds(e0, _CH)],
                       sem_wl)]

        fire_g = lambda t, ph: g_copies(t, ph, False)
        wait_g = lambda t, ph: [c.wait() for c in g_copies(t, ph, True)]
        fire_wb = lambda t, ph: wb_copies(t, ph, False)
        wait_wb = lambda t, ph: [c.wait() for c in wb_copies(t, ph, True)]

        fire_g(0, 0)
        fire_g(1, 1)
        wait_g(0, 0)
        fire_wb(0, 0)
        fire_g(2, 2)

        def body(t, _):
            ph = lax.rem(t, nbuf)
            wait_g(t, ph)
            fire_wb(t, ph)
            wait_wb(t - 1, lax.rem(t - 1, nbuf))
            fire_g(t + 2, lax.rem(t + 2, nbuf))
            return 0

        lax.fori_loop(1, n_iter - 2, body, 0)

        t = n_iter - 2
        wait_g(t, lax.rem(t, nbuf))
        fire_wb(t, lax.rem(t, nbuf))
        wait_wb(t - 1, lax.rem(t - 1, nbuf))
        t = n_iter - 1
        wait_g(t, lax.rem(t, nbuf))
        fire_wb(t, lax.rem(t, nbuf))
        wait_wb(t - 1, lax.rem(t - 1, nbuf))
        wait_wb(t, lax.rem(t, nbuf))

    return k(emb, last_update, idx1d)


_INV2PI = 0.15915494309189535
_COSC = (0.9999590249547727, -19.73094253387524, 64.67144342432282,
         -82.3908110654963, 45.62105237801009)


def _fast_cos(x):
    u = x * _INV2PI
    u = u - jnp.round(u)
    t = u * u
    r = jnp.float32(_COSC[4])
    for c in _COSC[3::-1]:
        r = r * t + jnp.float32(c)
    return r


def _tc_body(se_ref, ef_ref, ts_ref, lu_ref, em_ref,
             tw_ref, tb_ref, wqn_ref, wqt_ref, bq_ref,
             wkn_ref, wke_ref, wkt_ref,
             wvn_ref, wve_ref, wvt_ref, bv_ref,
             wo_ref, bo_ref, w1_ref, b1_ref, w2_ref, b2_ref,
             out_ref):
    f32 = jnp.float32
    dot = functools.partial(jnp.dot, preferred_element_type=f32)
    se = se_ref[...]
    ef = ef_ref[...]
    dt = (ts_ref[...] - lu_ref[...]).reshape(_BK // 128, 128)
    tw3 = tw_ref[...].reshape(1, 1, _DT)
    tb3 = tb_ref[...].reshape(1, 1, _DT)
    dt3 = dt[:, :, None]
    te = _fast_cos(dt3 * tw3 + tb3).reshape(_BK, _DT)
    tb = tb_ref[...]
    em = em_ref[...]

    te0 = jnp.cos(tb)
    cq = dot(te0, wqt_ref[...]) + bq_ref[...]
    q = (dot(em, wqn_ref[...]) + cq) * (1.0 / math.sqrt(128.0))

    se3 = se.reshape(_B, _K, _DN)
    ef3 = ef.reshape(_B, _K, _DE)
    te3 = te.reshape(_B, _K, _DT)

    ctxs = []
    for h in range(_H):
        hs = slice(h * 128, (h + 1) * 128)
        qh = q[:, hs]
        qkn = dot(qh, wkn_ref[hs, :])
        qke = dot(qh, wke_ref[hs, :])
        qkt = dot(qh, wkt_ref[hs, :])
        prod = se3 * qkn[:, None, :] + te3 * qkt[:, None, :]
        lg3 = (jnp.sum(prod, axis=-1, keepdims=True)
               + jnp.sum(ef3 * qke[:, None, :], axis=-1,
                         keepdims=True))
        m = jnp.max(lg3, axis=1, keepdims=True)
        p = jnp.exp(lg3 - m)
        a3 = p * (1.0 / jnp.sum(p, axis=1, keepdims=True))
        cn = jnp.sum(se3 * a3, axis=1)
        ce = jnp.sum(ef3 * a3, axis=1)
        ct = jnp.sum(te3 * a3, axis=1)
        ctx = (dot(cn, wvn_ref[:, hs]) + dot(ce, wve_ref[:, hs])
               + dot(ct, wvt_ref[:, hs]) + bv_ref[:, hs])
        ctxs.append(ctx)

    hb = (dot(ctxs[0], wo_ref[0:128, :]) + dot(ctxs[1], wo_ref[128:256, :])
          + bo_ref[...])
    x1 = dot(em, w1_ref[0:128, :]) + dot(hb, w1_ref[128:384, :]) + b1_ref[...]
    h1 = jnp.maximum(x1, 0.0)
    out_ref[...] = dot(h1, w2_ref[...]) + b2_ref[...]


def _tc_main(se, ef2, ts2, lu2, emb, tw2, tb2, wqnT, wqtT, bq2,
             wkn, wke, wkt, wvnT, wveT, wvtT, bv2,
             woT, bo2, w1T, b12, w2T, b22, part=0, interpret=False):
    full = lambda s: pl.BlockSpec(s, lambda i: (0, 0))
    grid = _NP // _B
    off = part * grid
    return pl.pallas_call(
        _tc_body,
        grid=(grid,),
        in_specs=[
            pl.BlockSpec((_BK, _DN), lambda i: (i, 0)),
            pl.BlockSpec((_BK, _DE), lambda i: (i + off, 0)),
            pl.BlockSpec((1, _BK // 128, 128), lambda i: (i + off, 0, 0)),
            pl.BlockSpec((1, _BK // 128, 128), lambda i: (i, 0, 0)),
            pl.BlockSpec((_B, _DN), lambda i: (i + off, 0)),
            full((1, _DT)), full((1, _DT)),
            full((_DN, _QD)), full((_DT, _QD)), full((1, _QD)),
            full((_QD, _DN)), full((_QD, _DE)), full((_QD, _DT)),
            full((_DN, _QD)), full((_DE, _QD)), full((_DT, _QD)),
            full((1, _QD)),
            full((_QD, _QD)), full((1, _QD)),
            full((_QD + _DN, _DN)), full((1, _DN)),
            full((_DN, _DN)), full((1, _DN)),
        ],
        out_specs=pl.BlockSpec((_B, _DN), lambda i: (i, 0)),
        out_shape=jax.ShapeDtypeStruct((_NP, _DN), jnp.float32),
        interpret=interpret,
    )(se, ef2, ts2, lu2, emb, tw2, tb2, wqnT, wqtT, bq2,
      wkn, wke, wkt, wvnT, wveT, wvtT, bv2, woT, bo2, w1T, b12, w2T, b22)


def kernel(emb, edge_feat, timestamp, last_update, src_idx, time_w, time_b,
           Wq, bq, Wk, bk, Wv, bv, Wo, bo, W1, b1, W2, b2):
    idx = src_idx.astype(jnp.int32)
    tw2 = time_w.reshape(1, _DT)
    tb2 = time_b.reshape(1, _DT)
    wqnT = Wq[:, :_DN].T
    wqtT = Wq[:, _DN:].T
    wkn = Wk[:, :_DN]
    wke = Wk[:, _DN:_DN + _DE]
    wkt = Wk[:, _DN + _DE:]
    wvnT = Wv[:, :_DN].T
    wveT = Wv[:, _DN:_DN + _DE].T
    wvtT = Wv[:, _DN + _DE:].T
    bq2 = bq.reshape(1, -1)
    bv2 = bv.reshape(1, -1)
    woT, bo2 = Wo.T, bo.reshape(1, -1)
    w1T, b12 = W1.T, b1.reshape(1, -1)
    w2T, b22 = W2.T, b2.reshape(1, -1)

    ts2 = timestamp.reshape(_N // _B, _BK // 128, 128)
    outs = []
    for p in range(_P):
        es = slice(p * _EP, (p + 1) * _EP)
        src_emb, lu_src = _sc_gather(emb, last_update, idx[es])
        lu2 = lu_src.reshape(_NP // _B, _BK // 128, 128)
        outs.append(_tc_main(src_emb, edge_feat, ts2, lu2, emb,
                             tw2, tb2, wqnT, wqtT, bq2,
                             wkn, wke, wkt, wvnT, wveT, wvtT, bv2,
                             woT, bo2, w1T, b12, w2T, b22, part=p))
    return jnp.concatenate(outs, axis=0)
